# baseline (speedup 1.0000x reference)
import functools

import jax
import jax.numpy as jnp
from jax.experimental import pallas as pl
from jax.experimental.pallas import tpu as pltpu


def _ce_body(x_hbm, out_ref, buf, sems, *, TB, nj, K):
    i = pl.program_id(0)
    base = i * nj * TB

    def _copy(b, slot):
        return pltpu.make_async_copy(
            x_hbm.at[pl.ds(base + b * TB, TB), :], buf.at[slot], sems.at[slot]
        )

    for k in range(K):
        _copy(k, k).start()

    def _step(b, acc):
        slot = jax.lax.rem(b, K)
        _copy(b, slot).wait()
        x = buf[slot]
        d = x[:, :1] - x[:, 1:2]
        sp = jnp.maximum(d, 0.0) + jnp.log1p(jnp.exp(-jnp.abs(d)))

        @pl.when(b + K < nj)
        def _refill():
            _copy(b + K, slot).start()

        return acc + jnp.sum(sp)

    acc = jax.lax.fori_loop(0, nj, _step, jnp.float32(0.0))
    out_ref[...] = jnp.full((1, 1, 1), acc, jnp.float32)


def kernel(logits):
    B, C = logits.shape

    TB = 16384
    K = 4
    nj = B // (2 * TB)

    partials = pl.pallas_call(
        functools.partial(_ce_body, TB=TB, nj=nj, K=K),
        out_shape=jax.ShapeDtypeStruct((2, 1, 1), jnp.float32),
        grid=(2,),
        in_specs=[pl.BlockSpec(memory_space=pltpu.MemorySpace.HBM)],
        out_specs=pl.BlockSpec((1, 1, 1), lambda i: (i, 0, 0)),
        scratch_shapes=[
            pltpu.VMEM((K, TB, C), jnp.float32),
            pltpu.SemaphoreType.DMA((K,)),
        ],
        compiler_params=pltpu.CompilerParams(
            dimension_semantics=("parallel",),
        ),
    )(logits)
    return partials.sum() * (1.0 / B)


# speedup vs baseline: 17.0454x; 17.0454x over previous
import functools

import jax
import jax.numpy as jnp
from jax.experimental import pallas as pl
from jax.experimental.pallas import tpu as pltpu


def _ce_body(x_ref, out_ref):
    j = pl.program_id(1)

    @pl.when(j == 0)
    def _init():
        out_ref[...] = jnp.zeros_like(out_ref)

    x = x_ref[...]
    d = x[:, 0, :] - x[:, 1, :]
    sp = jnp.maximum(d, 0.0) + jnp.log1p(jnp.exp(-jnp.abs(d)))
    out_ref[...] = out_ref[...] + jnp.sum(sp)


def kernel(logits):
    B, C = logits.shape
    G = B // 128
    dense = logits.reshape(G, 128, C).transpose(0, 2, 1)

    GB = 1024
    nj = G // (2 * GB)
    grid = (2, nj)

    partials = pl.pallas_call(
        _ce_body,
        out_shape=jax.ShapeDtypeStruct((2, 1, 1), jnp.float32),
        grid=grid,
        in_specs=[pl.BlockSpec((GB, C, 128), lambda i, j: (i * nj + j, 0, 0))],
        out_specs=pl.BlockSpec((1, 1, 1), lambda i, j: (i, 0, 0)),
        compiler_params=pltpu.CompilerParams(
            dimension_semantics=("parallel", "arbitrary"),
        ),
    )(dense)
    return partials.sum() * (1.0 / B)


# speedup vs baseline: 49.9167x; 2.9284x over previous
import functools

import jax
import jax.numpy as jnp
from jax.experimental import pallas as pl
from jax.experimental.pallas import tpu as pltpu


def _ce_body(x_ref, out_ref):
    j = pl.program_id(1)

    @pl.when(j == 0)
    def _init():
        out_ref[...] = jnp.zeros_like(out_ref)

    x = x_ref[...]
    r = pltpu.roll(x, x.shape[0] - 1, 0)
    d = x - r
    sp = jnp.maximum(d, 0.0) + jnp.log1p(jnp.exp(-jnp.abs(d)))
    row = jax.lax.broadcasted_iota(jnp.int32, x.shape, 0)
    sp = jnp.where(row % 2 == 0, sp, 0.0)
    out_ref[...] = out_ref[...] + jnp.sum(sp)


def kernel(logits):
    B, C = logits.shape
    G = B // 128
    dense = logits.reshape(G, 128, C).transpose(0, 2, 1).reshape(C * G, 128)

    RB = 2048
    nj = (C * G) // (2 * RB)
    grid = (2, nj)

    partials = pl.pallas_call(
        _ce_body,
        out_shape=jax.ShapeDtypeStruct((2, 1, 1), jnp.float32),
        grid=grid,
        in_specs=[pl.BlockSpec((RB, 128), lambda i, j: (i * nj + j, 0))],
        out_specs=pl.BlockSpec((1, 1, 1), lambda i, j: (i, 0, 0)),
        compiler_params=pltpu.CompilerParams(
            dimension_semantics=("parallel", "arbitrary"),
        ),
    )(dense)
    return partials.sum() * (1.0 / B)
